# baseline (speedup 1.0000x reference)
import functools

import jax
import jax.numpy as jnp
from jax.experimental import pallas as pl
from jax.experimental.pallas import tpu as pltpu

NUM_FILTERS = 2
N = 2048
D = 128
EPS = 1e-5

NBLK = 8
BLK = (NUM_FILTERS * N) // NBLK
HALF = NBLK // 2
NSTAGE = 4


def _layer_norm(x, g, b):
    m = jnp.mean(x, axis=-1, keepdims=True)
    v = jnp.mean((x - m) ** 2, axis=-1, keepdims=True)
    return (x - m) / jnp.sqrt(v + EPS) * g + b


def _body(x_ref, f_hbm, w1_ref, b1_ref, g1_ref, be1_ref,
          w2_ref, b2_ref, g2_ref, be2_ref,
          w3_ref, b3_ref, g3_ref, be3_ref, o_ref,
          stage, fb_scr, cc_scr, h_scr, sems):

    def copy(i):
        return pltpu.make_async_copy(
            f_hbm.at[pl.ds(i * BLK, BLK), :],
            stage.at[i % NSTAGE],
            sems.at[i],
        )

    for i in range(NSTAGE):
        copy(i).start()

    xb = x_ref[...].astype(jnp.bfloat16)
    w1 = w1_ref[...].astype(jnp.bfloat16)

    for i in range(NBLK):
        copy(i).wait()
        fb = stage[i % NSTAGE].astype(jnp.bfloat16)
        if i + NSTAGE < NBLK:
            copy(i + NSTAGE).start()
        fb_scr[pl.ds(i * BLK, BLK), :] = fb
        part = jnp.dot(fb, xb, preferred_element_type=jnp.float32)
        f, r = divmod(i, HALF)
        cc_scr[pl.ds(r * BLK, BLK), pl.ds(f * D, D)] = part.astype(jnp.bfloat16)
        if f == 1:
            rows = pl.ds(r * BLK, BLK)
            z = jnp.dot(cc_scr[rows, :], w1,
                        preferred_element_type=jnp.float32) + b1_ref[...]
            h1 = _layer_norm(jax.nn.relu(z), g1_ref[...], be1_ref[...])
            h_scr[rows, :] = h1.astype(jnp.bfloat16)

    def dense_relu(w_ref, b_ref):
        z = jnp.dot(cc_scr[...], w_ref[...].astype(jnp.bfloat16),
                    preferred_element_type=jnp.float32) + b_ref[...]
        return jax.nn.relu(z)

    def conv_layer(hb, w_ref, b_ref):
        for f in range(NUM_FILTERS):
            part = jnp.dot(fb_scr[pl.ds(f * N, N), :], hb,
                           preferred_element_type=jnp.float32)
            cc_scr[:, pl.ds(f * D, D)] = part.astype(jnp.bfloat16)
        return dense_relu(w_ref, b_ref)

    h = conv_layer(h_scr[...], w2_ref, b2_ref)
    h = _layer_norm(h, g2_ref[...], be2_ref[...])
    h = conv_layer(h.astype(jnp.bfloat16), w3_ref, b3_ref)
    out = _layer_norm(x_ref[...] + h, g3_ref[...], be3_ref[...])
    o_ref[...] = jax.nn.relu(out)


@functools.partial(jax.jit)
def _run(X, filt, W1, b1, g1, be1, W2, b2, g2, be2, W3, b3, g3, be3):
    x2 = X.reshape(N, D)
    f2 = filt.reshape(NUM_FILTERS * N, N)
    vecs = [v.reshape(1, D) for v in (b1, g1, be1, b2, g2, be2, b3, g3, be3)]
    b1r, g1r, be1r, b2r, g2r, be2r, b3r, g3r, be3r = vecs
    vspec = pl.BlockSpec(memory_space=pltpu.MemorySpace.VMEM)
    out = pl.pallas_call(
        _body,
        in_specs=[
            vspec,
            pl.BlockSpec(memory_space=pltpu.MemorySpace.HBM),
            vspec, vspec, vspec, vspec,
            vspec, vspec, vspec, vspec,
            vspec, vspec, vspec, vspec,
        ],
        out_specs=vspec,
        out_shape=jax.ShapeDtypeStruct((N, D), jnp.float32),
        scratch_shapes=[
            pltpu.VMEM((NSTAGE, BLK, N), jnp.float32),
            pltpu.VMEM((NUM_FILTERS * N, N), jnp.bfloat16),
            pltpu.VMEM((N, NUM_FILTERS * D), jnp.bfloat16),
            pltpu.VMEM((N, D), jnp.bfloat16),
            pltpu.SemaphoreType.DMA((NBLK,)),
        ],
        compiler_params=pltpu.CompilerParams(
            vmem_limit_bytes=100 * 1024 * 1024,
        ),
    )(x2, f2, W1, b1r, g1r, be1r, W2, b2r, g2r, be2r, W3, b3r, g3r, be3r)
    return out.reshape(1, N, D)


def kernel(X, graph_conv_filters_input, W1, b1, g1, be1,
           W2, b2, g2, be2, W3, b3, g3, be3):
    return _run(X, graph_conv_filters_input, W1, b1, g1, be1,
                W2, b2, g2, be2, W3, b3, g3, be3)


# speedup vs baseline: 1.0474x; 1.0474x over previous
import functools

import jax
import jax.numpy as jnp
from jax.experimental import pallas as pl
from jax.experimental.pallas import tpu as pltpu

NUM_FILTERS = 2
N = 2048
D = 128
EPS = 1e-5

NBLK = 8
BLK = (NUM_FILTERS * N) // NBLK
HALF = NBLK // 2
NSTAGE = 4


def _layer_norm(x, g, b):
    m = jnp.mean(x, axis=-1, keepdims=True)
    v = jnp.mean((x - m) ** 2, axis=-1, keepdims=True)
    return (x - m) / jnp.sqrt(v + EPS) * g + b


def _body(x_ref, f_hbm, w1_ref, b1_ref, g1_ref, be1_ref,
          w2_ref, b2_ref, g2_ref, be2_ref,
          w3_ref, b3_ref, g3_ref, be3_ref, o_ref,
          stage, fb_scr, cc_scr, sems):

    def copy(i):
        return pltpu.make_async_copy(
            f_hbm.at[pl.ds(i * BLK, BLK), :],
            stage.at[i % NSTAGE],
            sems.at[i],
        )

    for i in range(NSTAGE):
        copy(i).start()

    xb = x_ref[...].astype(jnp.bfloat16)

    for i in range(NBLK):
        copy(i).wait()
        fb = stage[i % NSTAGE].astype(jnp.bfloat16)
        if i + NSTAGE < NBLK:
            copy(i + NSTAGE).start()
        fb_scr[pl.ds(i * BLK, BLK), :] = fb
        part = jnp.dot(fb, xb, preferred_element_type=jnp.float32)
        f, r = divmod(i, HALF)
        cc_scr[pl.ds(r * BLK, BLK), pl.ds(f * D, D)] = part.astype(jnp.bfloat16)

    def dense_relu(w_ref, b_ref):
        z = jnp.dot(cc_scr[...], w_ref[...].astype(jnp.bfloat16),
                    preferred_element_type=jnp.float32) + b_ref[...]
        return jax.nn.relu(z)

    def conv_layer(h, w_ref, b_ref):
        hb = h.astype(jnp.bfloat16)
        for f in range(NUM_FILTERS):
            part = jnp.dot(fb_scr[pl.ds(f * N, N), :], hb,
                           preferred_element_type=jnp.float32)
            cc_scr[:, pl.ds(f * D, D)] = part.astype(jnp.bfloat16)
        return dense_relu(w_ref, b_ref)

    h = dense_relu(w1_ref, b1_ref)
    h = _layer_norm(h, g1_ref[...], be1_ref[...])
    h = conv_layer(h, w2_ref, b2_ref)
    h = _layer_norm(h, g2_ref[...], be2_ref[...])
    h = conv_layer(h, w3_ref, b3_ref)
    out = _layer_norm(x_ref[...] + h, g3_ref[...], be3_ref[...])
    o_ref[...] = jax.nn.relu(out)


@functools.partial(jax.jit)
def _run(X, filt, W1, b1, g1, be1, W2, b2, g2, be2, W3, b3, g3, be3):
    x2 = X.reshape(N, D)
    f2 = filt.reshape(NUM_FILTERS * N, N)
    vecs = [v.reshape(1, D) for v in (b1, g1, be1, b2, g2, be2, b3, g3, be3)]
    b1r, g1r, be1r, b2r, g2r, be2r, b3r, g3r, be3r = vecs
    vspec = pl.BlockSpec(memory_space=pltpu.MemorySpace.VMEM)
    out = pl.pallas_call(
        _body,
        in_specs=[
            vspec,
            pl.BlockSpec(memory_space=pltpu.MemorySpace.HBM),
            vspec, vspec, vspec, vspec,
            vspec, vspec, vspec, vspec,
            vspec, vspec, vspec, vspec,
        ],
        out_specs=vspec,
        out_shape=jax.ShapeDtypeStruct((N, D), jnp.float32),
        scratch_shapes=[
            pltpu.VMEM((NSTAGE, BLK, N), jnp.float32),
            pltpu.VMEM((NUM_FILTERS * N, N), jnp.bfloat16),
            pltpu.VMEM((N, NUM_FILTERS * D), jnp.bfloat16),
            pltpu.SemaphoreType.DMA((NBLK,)),
        ],
        compiler_params=pltpu.CompilerParams(
            vmem_limit_bytes=100 * 1024 * 1024,
        ),
    )(x2, f2, W1, b1r, g1r, be1r, W2, b2r, g2r, be2r, W3, b3r, g3r, be3r)
    return out.reshape(1, N, D)


def kernel(X, graph_conv_filters_input, W1, b1, g1, be1,
           W2, b2, g2, be2, W3, b3, g3, be3):
    return _run(X, graph_conv_filters_input, W1, b1, g1, be1,
                W2, b2, g2, be2, W3, b3, g3, be3)
